# baseline (speedup 1.0000x reference)
import functools

import numpy as np
import jax
import jax.numpy as jnp
from jax import lax
from jax.experimental import pallas as pl
from jax.experimental.pallas import tpu as pltpu
from jax.experimental.pallas import tpu_sc as plsc

_NC, _NS = 2, 16
_NW = _NC * _NS
_CH = 80


def _sc_word_gather(table, idx):
    (bs,) = idx.shape
    v, d = table.shape
    b_per_w = bs // _NW
    n_ch = b_per_w // _CH
    idx3 = idx.reshape(_NW, n_ch, _CH)
    mesh = plsc.VectorSubcoreMesh(core_axis_name="c", subcore_axis_name="s")

    @functools.partial(
        pl.kernel,
        mesh=mesh,
        compiler_params=pltpu.CompilerParams(use_tc_tiling_on_sc=True),
        out_type=jax.ShapeDtypeStruct((bs, d), jnp.float32),
        scratch_types=[
            pltpu.VMEM((n_ch, _CH), jnp.int32),
            pltpu.VMEM((2, _CH, d), jnp.float32),
            pltpu.SemaphoreType.DMA,
            pltpu.SemaphoreType.DMA,
        ],
    )
    def gather_kernel(table_hbm, idx_hbm, out_hbm, idx_v, rows_v, sem0, sem1):
        wid = lax.axis_index("s") * _NC + lax.axis_index("c")
        base = wid * b_per_w
        pltpu.sync_copy(idx_hbm.at[wid], idx_v)
        sems = (sem0, sem1)
        handles = {}

        def start(c):
            buf = c % 2
            handles[c] = pltpu.async_copy(
                table_hbm.at[idx_v.at[c]], rows_v.at[buf], sems[buf]
            )

        start(0)
        for c in range(n_ch):
            handles[c].wait()
            if c + 1 < n_ch:
                start(c + 1)
            pltpu.sync_copy(
                rows_v.at[c % 2], out_hbm.at[pl.ds(base + c * _CH, _CH)]
            )

    return gather_kernel(table, idx3)


def _conv_matrix(conv_w):
    wr = jnp.transpose(conv_w, (1, 0, 2)).reshape(16, 4, 64, 3)
    sel = np.zeros((3, 16, 16), np.float32)
    for k in range(3):
        for tau in range(16):
            t = tau - k + 1
            if 0 <= t < 16:
                sel[k, tau, t] = 1.0
    a = jnp.einsum("pjok,kab->pjabo", wr, jnp.asarray(sel))
    return a.reshape(1024, 1024)


_NB = 512


def _char_conv_tc(cin, ct, afull, bias):
    bs = cin.shape[0]
    grid = bs // _NB

    def body(cin_ref, ct_ref, a_ref, b_ref, out_ref):
        ctv = ct_ref[...]
        parts = []
        for p in range(16):
            col = cin_ref[:, p : p + 1]
            oh = (col == lax.broadcasted_iota(jnp.int32, (_NB, 256), 1)).astype(
                jnp.float32
            )
            parts.append(jnp.dot(oh, ctv, preferred_element_type=jnp.float32))
        e = jnp.concatenate(parts, axis=1)
        y = jnp.dot(e, a_ref[...], preferred_element_type=jnp.float32)
        acc = y[:, 0:128]
        for t8 in range(1, 8):
            acc = jnp.maximum(acc, y[:, 128 * t8 : 128 * (t8 + 1)])
        pooled = jnp.maximum(acc[:, 0:64], acc[:, 64:128]) + b_ref[...]
        out_ref[...] = pooled

    return pl.pallas_call(
        body,
        grid=(grid,),
        in_specs=[
            pl.BlockSpec((_NB, 16), lambda i: (i, 0)),
            pl.BlockSpec((256, 64), lambda i: (0, 0)),
            pl.BlockSpec((1024, 1024), lambda i: (0, 0)),
            pl.BlockSpec((1, 64), lambda i: (0, 0)),
        ],
        out_specs=pl.BlockSpec((_NB, 64), lambda i: (i, 0)),
        out_shape=jax.ShapeDtypeStruct((bs, 64), jnp.float32),
    )(cin, ct, afull, bias)


_AB = 16


def _assemble_tc(wemb, cemb, b, s):
    ns = _AB * s
    grid = b // _AB

    def body(w_ref, c_ref, out_ref):
        out_ref[:, :, 0:128] = w_ref[...].reshape(_AB, s, 128)
        out_ref[:, :, 128:192] = c_ref[...].reshape(_AB, s, 64)

    return pl.pallas_call(
        body,
        grid=(grid,),
        in_specs=[
            pl.BlockSpec((ns, 128), lambda i: (i, 0)),
            pl.BlockSpec((ns, 64), lambda i: (i, 0)),
        ],
        out_specs=pl.BlockSpec((_AB, s, 192), lambda i: (i, 0, 0)),
        out_shape=jax.ShapeDtypeStruct((b, s, 192), jnp.float32),
    )(wemb, cemb)


def kernel(word_inputs, char_inputs, word_table, char_table, conv_w, conv_b):
    b, s = word_inputs.shape
    w = char_inputs.shape[-1]
    idx = word_inputs.reshape(-1).astype(jnp.int32)
    wemb = _sc_word_gather(word_table, idx)
    cin = char_inputs.reshape(-1, w).astype(jnp.int32)
    afull = _conv_matrix(conv_w)
    bias = conv_b.reshape(1, 64)
    cemb = _char_conv_tc(cin, char_table, afull, bias)
    return jnp.concatenate([wemb, cemb], axis=1).reshape(b, s, 192)


# speedup vs baseline: 1.0326x; 1.0326x over previous
import functools

import numpy as np
import jax
import jax.numpy as jnp
from jax import lax
from jax.experimental import pallas as pl
from jax.experimental.pallas import tpu as pltpu
from jax.experimental.pallas import tpu_sc as plsc

_NC, _NS = 2, 16
_NW = _NC * _NS
_CH = 80


def _sc_word_gather(table, idx):
    (bs,) = idx.shape
    v, d = table.shape
    b_per_w = bs // _NW
    n_ch = b_per_w // _CH
    idx3 = idx.reshape(_NW, n_ch, _CH)
    mesh = plsc.VectorSubcoreMesh(core_axis_name="c", subcore_axis_name="s")

    @functools.partial(
        pl.kernel,
        mesh=mesh,
        compiler_params=pltpu.CompilerParams(use_tc_tiling_on_sc=True),
        out_type=jax.ShapeDtypeStruct((bs, d), jnp.float32),
        scratch_types=[
            pltpu.VMEM((n_ch, _CH), jnp.int32),
            pltpu.VMEM((2, _CH, d), jnp.float32),
            pltpu.SemaphoreType.DMA,
            pltpu.SemaphoreType.DMA,
        ],
    )
    def gather_kernel(table_hbm, idx_hbm, out_hbm, idx_v, rows_v, sem0, sem1):
        wid = lax.axis_index("s") * _NC + lax.axis_index("c")
        base = wid * b_per_w
        pltpu.sync_copy(idx_hbm.at[wid], idx_v)
        sems = (sem0, sem1)
        handles = {}

        def start(c):
            buf = c % 2
            handles[c] = pltpu.async_copy(
                table_hbm.at[idx_v.at[c]], rows_v.at[buf], sems[buf]
            )

        start(0)
        for c in range(n_ch):
            handles[c].wait()
            if c + 1 < n_ch:
                start(c + 1)
            pltpu.sync_copy(
                rows_v.at[c % 2], out_hbm.at[pl.ds(base + c * _CH, _CH)]
            )

    return gather_kernel(table, idx3)


def _conv_matrix(conv_w):
    wt = jnp.transpose(conv_w, (1, 0, 2))
    rowmod = (np.arange(1024) % 16).astype(np.int32)[:, None]
    colgrp = (np.arange(1024) // 64).astype(np.int32)[None, :]
    diff = jnp.asarray(rowmod - colgrp)
    a = jnp.zeros((1024, 1024), jnp.float32)
    for k in range(3):
        wexp = jnp.tile(jnp.repeat(wt[:, :, k], 16, axis=0), (1, 16))
        a = a + jnp.where(diff == (k - 1), wexp, 0.0)
    return a


_NB = 512


def _char_conv_tc(cin, ct, afull, bias):
    bs = cin.shape[0]
    grid = bs // _NB

    def body(cin_ref, ct_ref, a_ref, b_ref, out_ref):
        ctv = ct_ref[...]
        parts = []
        for p in range(16):
            col = cin_ref[:, p : p + 1]
            oh = (col == lax.broadcasted_iota(jnp.int32, (_NB, 256), 1)).astype(
                jnp.float32
            )
            parts.append(jnp.dot(oh, ctv, preferred_element_type=jnp.float32))
        e = jnp.concatenate(parts, axis=1)
        y = jnp.dot(e, a_ref[...], preferred_element_type=jnp.float32)
        acc = y[:, 0:128]
        for t8 in range(1, 8):
            acc = jnp.maximum(acc, y[:, 128 * t8 : 128 * (t8 + 1)])
        pooled = jnp.maximum(acc[:, 0:64], acc[:, 64:128]) + b_ref[...]
        out_ref[...] = pooled

    return pl.pallas_call(
        body,
        grid=(grid,),
        in_specs=[
            pl.BlockSpec((_NB, 16), lambda i: (i, 0)),
            pl.BlockSpec((256, 64), lambda i: (0, 0)),
            pl.BlockSpec((1024, 1024), lambda i: (0, 0)),
            pl.BlockSpec((1, 64), lambda i: (0, 0)),
        ],
        out_specs=pl.BlockSpec((_NB, 64), lambda i: (i, 0)),
        out_shape=jax.ShapeDtypeStruct((bs, 64), jnp.float32),
    )(cin, ct, afull, bias)


_AB = 16


def _assemble_tc(wemb, cemb, b, s):
    ns = _AB * s
    grid = b // _AB

    def body(w_ref, c_ref, out_ref):
        out_ref[:, :, 0:128] = w_ref[...].reshape(_AB, s, 128)
        out_ref[:, :, 128:192] = c_ref[...].reshape(_AB, s, 64)

    return pl.pallas_call(
        body,
        grid=(grid,),
        in_specs=[
            pl.BlockSpec((ns, 128), lambda i: (i, 0)),
            pl.BlockSpec((ns, 64), lambda i: (i, 0)),
        ],
        out_specs=pl.BlockSpec((_AB, s, 192), lambda i: (i, 0, 0)),
        out_shape=jax.ShapeDtypeStruct((b, s, 192), jnp.float32),
    )(wemb, cemb)


def kernel(word_inputs, char_inputs, word_table, char_table, conv_w, conv_b):
    b, s = word_inputs.shape
    w = char_inputs.shape[-1]
    idx = word_inputs.reshape(-1).astype(jnp.int32)
    wemb = _sc_word_gather(word_table, idx)
    cin = char_inputs.reshape(-1, w).astype(jnp.int32)
    afull = _conv_matrix(conv_w)
    bias = conv_b.reshape(1, 64)
    cemb = _char_conv_tc(cin, char_table, afull, bias)
    return _assemble_tc(wemb, cemb, b, s)


# speedup vs baseline: 1.0902x; 1.0558x over previous
import functools

import numpy as np
import jax
import jax.numpy as jnp
from jax import lax
from jax.experimental import pallas as pl
from jax.experimental.pallas import tpu as pltpu
from jax.experimental.pallas import tpu_sc as plsc

_NC, _NS = 2, 16
_NW = _NC * _NS
_CH = 80


def _sc_word_gather(table, idx):
    (bs,) = idx.shape
    v, d = table.shape
    b_per_w = bs // _NW
    n_ch = b_per_w // _CH
    idx3 = idx.reshape(_NW, n_ch, _CH)
    mesh = plsc.VectorSubcoreMesh(core_axis_name="c", subcore_axis_name="s")

    @functools.partial(
        pl.kernel,
        mesh=mesh,
        compiler_params=pltpu.CompilerParams(use_tc_tiling_on_sc=True),
        out_type=jax.ShapeDtypeStruct((bs, d), jnp.float32),
        scratch_types=[
            pltpu.VMEM((n_ch, _CH), jnp.int32),
            pltpu.VMEM((2, _CH, d), jnp.float32),
            pltpu.SemaphoreType.DMA,
            pltpu.SemaphoreType.DMA,
        ],
    )
    def gather_kernel(table_hbm, idx_hbm, out_hbm, idx_v, rows_v, sem0, sem1):
        wid = lax.axis_index("s") * _NC + lax.axis_index("c")
        base = wid * b_per_w
        pltpu.sync_copy(idx_hbm.at[wid], idx_v)
        sems = (sem0, sem1)
        handles = {}

        def start(c):
            buf = c % 2
            handles[c] = pltpu.async_copy(
                table_hbm.at[idx_v.at[c]], rows_v.at[buf], sems[buf]
            )

        start(0)
        for c in range(n_ch):
            handles[c].wait()
            if c + 1 < n_ch:
                start(c + 1)
            pltpu.sync_copy(
                rows_v.at[c % 2], out_hbm.at[pl.ds(base + c * _CH, _CH)]
            )

    return gather_kernel(table, idx3)


def _conv_matrix(conv_w):
    wt = jnp.transpose(conv_w, (1, 0, 2))
    rowmod = (np.arange(1024) % 16).astype(np.int32)[:, None]
    colgrp = (np.arange(1024) // 64).astype(np.int32)[None, :]
    diff = jnp.asarray(rowmod - colgrp)
    a = jnp.zeros((1024, 1024), jnp.float32)
    for k in range(3):
        wexp = jnp.tile(jnp.repeat(wt[:, :, k], 16, axis=0), (1, 16))
        a = a + jnp.where(diff == (k - 1), wexp, 0.0)
    return a


_NB = 512


def _char_conv_tc(cin, ct, afull, bias):
    bs = cin.shape[0]
    grid = bs // _NB

    def body(cin_ref, ct_ref, a_ref, b_ref, out_ref):
        ctv = ct_ref[...]
        parts = []
        for p in range(16):
            col = cin_ref[:, p : p + 1]
            oh = (col == lax.broadcasted_iota(jnp.int32, (_NB, 256), 1)).astype(
                jnp.float32
            )
            parts.append(jnp.dot(oh, ctv, preferred_element_type=jnp.float32))
        e = jnp.concatenate(parts, axis=1)
        y = jnp.dot(e, a_ref[...], preferred_element_type=jnp.float32)
        acc = y[:, 0:128]
        for t8 in range(1, 8):
            acc = jnp.maximum(acc, y[:, 128 * t8 : 128 * (t8 + 1)])
        pooled = jnp.maximum(acc[:, 0:64], acc[:, 64:128]) + b_ref[...]
        out_ref[...] = pooled

    return pl.pallas_call(
        body,
        grid=(grid,),
        in_specs=[
            pl.BlockSpec((_NB, 16), lambda i: (i, 0)),
            pl.BlockSpec((256, 64), lambda i: (0, 0)),
            pl.BlockSpec((1024, 1024), lambda i: (0, 0)),
            pl.BlockSpec((1, 64), lambda i: (0, 0)),
        ],
        out_specs=pl.BlockSpec((_NB, 64), lambda i: (i, 0)),
        out_shape=jax.ShapeDtypeStruct((bs, 64), jnp.float32),
    )(cin, ct, afull, bias)


_AB = 64


def _assemble_tc(wemb, cemb, b, s):
    ns = _AB * s
    grid = b // _AB

    def body(w_ref, c_ref, out_ref):
        out_ref[:, :, 0:128] = w_ref[...].reshape(_AB, s, 128)
        out_ref[:, :, 128:192] = c_ref[...].reshape(_AB, s, 64)

    return pl.pallas_call(
        body,
        grid=(grid,),
        in_specs=[
            pl.BlockSpec((ns, 128), lambda i: (i, 0)),
            pl.BlockSpec((ns, 64), lambda i: (i, 0)),
        ],
        out_specs=pl.BlockSpec((_AB, s, 192), lambda i: (i, 0, 0)),
        out_shape=jax.ShapeDtypeStruct((b, s, 192), jnp.float32),
    )(wemb, cemb)


def kernel(word_inputs, char_inputs, word_table, char_table, conv_w, conv_b):
    b, s = word_inputs.shape
    w = char_inputs.shape[-1]
    idx = word_inputs.reshape(-1).astype(jnp.int32)
    wemb = _sc_word_gather(word_table, idx)
    cin = char_inputs.reshape(-1, w).astype(jnp.int32)
    afull = _conv_matrix(conv_w)
    bias = conv_b.reshape(1, 64)
    cemb = _char_conv_tc(cin, char_table, afull, bias)
    return _assemble_tc(wemb, cemb, b, s)


# speedup vs baseline: 1.0919x; 1.0016x over previous
import functools

import numpy as np
import jax
import jax.numpy as jnp
from jax import lax
from jax.experimental import pallas as pl
from jax.experimental.pallas import tpu as pltpu
from jax.experimental.pallas import tpu_sc as plsc

_NC, _NS = 2, 16
_NW = _NC * _NS
_CH = 80


def _sc_word_gather(table, idx):
    (bs,) = idx.shape
    v, d = table.shape
    b_per_w = bs // _NW
    n_ch = b_per_w // _CH
    idx3 = idx.reshape(_NW, n_ch, _CH)
    mesh = plsc.VectorSubcoreMesh(core_axis_name="c", subcore_axis_name="s")

    @functools.partial(
        pl.kernel,
        mesh=mesh,
        compiler_params=pltpu.CompilerParams(use_tc_tiling_on_sc=True),
        out_type=jax.ShapeDtypeStruct((bs, d), jnp.float32),
        scratch_types=[
            pltpu.VMEM((n_ch, _CH), jnp.int32),
            pltpu.VMEM((2, _CH, d), jnp.float32),
            pltpu.SemaphoreType.DMA,
            pltpu.SemaphoreType.DMA,
        ],
    )
    def gather_kernel(table_hbm, idx_hbm, out_hbm, idx_v, rows_v, sem0, sem1):
        wid = lax.axis_index("s") * _NC + lax.axis_index("c")
        base = wid * b_per_w
        pltpu.sync_copy(idx_hbm.at[wid], idx_v)
        sems = (sem0, sem1)
        handles = {}

        def start(c):
            buf = c % 2
            handles[c] = pltpu.async_copy(
                table_hbm.at[idx_v.at[c]], rows_v.at[buf], sems[buf]
            )

        start(0)
        for c in range(n_ch):
            handles[c].wait()
            if c + 1 < n_ch:
                start(c + 1)
            pltpu.sync_copy(
                rows_v.at[c % 2], out_hbm.at[pl.ds(base + c * _CH, _CH)]
            )

    return gather_kernel(table, idx3)


def _conv_matrix(conv_w):
    wt = jnp.transpose(conv_w, (1, 0, 2))
    rowmod = (np.arange(1024) % 16).astype(np.int32)[:, None]
    colgrp = (np.arange(1024) // 64).astype(np.int32)[None, :]
    diff = jnp.asarray(rowmod - colgrp)
    a = jnp.zeros((1024, 1024), jnp.float32)
    for k in range(3):
        wexp = jnp.tile(jnp.repeat(wt[:, :, k], 16, axis=0), (1, 16))
        a = a + jnp.where(diff == (k - 1), wexp, 0.0)
    return a


_NB = 512


def _char_conv_tc(cin, ct, afull, bias):
    bs = cin.shape[0]
    grid = bs // _NB

    def body(cin_ref, ct_ref, a_ref, b_ref, out_ref):
        ctv = ct_ref[...]
        parts = []
        for p in range(16):
            col = cin_ref[:, p : p + 1]
            oh = (col == lax.broadcasted_iota(jnp.int32, (_NB, 256), 1)).astype(
                jnp.float32
            )
            parts.append(jnp.dot(oh, ctv, preferred_element_type=jnp.float32))
        e = jnp.concatenate(parts, axis=1)
        y = jnp.dot(e, a_ref[...], preferred_element_type=jnp.float32)
        acc = y[:, 0:128]
        for t8 in range(1, 8):
            acc = jnp.maximum(acc, y[:, 128 * t8 : 128 * (t8 + 1)])
        pooled = jnp.maximum(acc[:, 0:64], acc[:, 64:128]) + b_ref[...]
        out_ref[...] = pooled

    return pl.pallas_call(
        body,
        grid=(grid,),
        in_specs=[
            pl.BlockSpec((_NB, 16), lambda i: (i, 0)),
            pl.BlockSpec((256, 64), lambda i: (0, 0)),
            pl.BlockSpec((1024, 1024), lambda i: (0, 0)),
            pl.BlockSpec((1, 64), lambda i: (0, 0)),
        ],
        out_specs=pl.BlockSpec((_NB, 64), lambda i: (i, 0)),
        out_shape=jax.ShapeDtypeStruct((bs, 64), jnp.float32),
    )(cin, ct, afull, bias)


_AB = 128


def _assemble_tc(wemb, cemb, b, s):
    ns = _AB * s
    grid = b // _AB

    def body(w_ref, c_ref, out_ref):
        out_ref[:, :, 0:128] = w_ref[...].reshape(_AB, s, 128)
        out_ref[:, :, 128:192] = c_ref[...].reshape(_AB, s, 64)

    return pl.pallas_call(
        body,
        grid=(grid,),
        in_specs=[
            pl.BlockSpec((ns, 128), lambda i: (i, 0)),
            pl.BlockSpec((ns, 64), lambda i: (i, 0)),
        ],
        out_specs=pl.BlockSpec((_AB, s, 192), lambda i: (i, 0, 0)),
        out_shape=jax.ShapeDtypeStruct((b, s, 192), jnp.float32),
    )(wemb, cemb)


def kernel(word_inputs, char_inputs, word_table, char_table, conv_w, conv_b):
    b, s = word_inputs.shape
    w = char_inputs.shape[-1]
    idx = word_inputs.reshape(-1).astype(jnp.int32)
    wemb = _sc_word_gather(word_table, idx)
    cin = char_inputs.reshape(-1, w).astype(jnp.int32)
    afull = _conv_matrix(conv_w)
    bias = conv_b.reshape(1, 64)
    cemb = _char_conv_tc(cin, char_table, afull, bias)
    return _assemble_tc(wemb, cemb, b, s)


# speedup vs baseline: 1.1352x; 1.0397x over previous
import functools

import numpy as np
import jax
import jax.numpy as jnp
from jax import lax
from jax.experimental import pallas as pl
from jax.experimental.pallas import tpu as pltpu
from jax.experimental.pallas import tpu_sc as plsc

_NC, _NS = 2, 16
_NW = _NC * _NS
_CH = 80


def _sc_word_gather(table, idx):
    (bs,) = idx.shape
    v, d = table.shape
    b_per_w = bs // _NW
    n_ch = b_per_w // _CH
    idx3 = idx.reshape(_NW, n_ch, _CH)
    mesh = plsc.VectorSubcoreMesh(core_axis_name="c", subcore_axis_name="s")

    @functools.partial(
        pl.kernel,
        mesh=mesh,
        compiler_params=pltpu.CompilerParams(use_tc_tiling_on_sc=True),
        out_type=jax.ShapeDtypeStruct((bs, d), jnp.float32),
        scratch_types=[
            pltpu.VMEM((n_ch, _CH), jnp.int32),
            pltpu.VMEM((2, _CH, d), jnp.float32),
            pltpu.SemaphoreType.DMA,
            pltpu.SemaphoreType.DMA,
        ],
    )
    def gather_kernel(table_hbm, idx_hbm, out_hbm, idx_v, rows_v, sem0, sem1):
        wid = lax.axis_index("s") * _NC + lax.axis_index("c")
        base = wid * b_per_w
        pltpu.sync_copy(idx_hbm.at[wid], idx_v)
        sems = (sem0, sem1)
        handles = {}

        def start(c):
            buf = c % 2
            handles[c] = pltpu.async_copy(
                table_hbm.at[idx_v.at[c]], rows_v.at[buf], sems[buf]
            )

        start(0)
        for c in range(n_ch):
            handles[c].wait()
            if c + 1 < n_ch:
                start(c + 1)
            pltpu.sync_copy(
                rows_v.at[c % 2], out_hbm.at[pl.ds(base + c * _CH, _CH)]
            )

    return gather_kernel(table, idx3)


def _conv_matrix(conv_w):
    wt = jnp.transpose(conv_w, (1, 0, 2))
    rowmod = (np.arange(1024) % 16).astype(np.int32)[:, None]
    colgrp = (np.arange(1024) // 64).astype(np.int32)[None, :]
    diff = jnp.asarray(rowmod - colgrp)
    a = jnp.zeros((1024, 1024), jnp.float32)
    for k in range(3):
        wexp = jnp.tile(jnp.repeat(wt[:, :, k], 16, axis=0), (1, 16))
        a = a + jnp.where(diff == (k - 1), wexp, 0.0)
    return a


_NB = 1024


def _char_conv_tc(cin, ct, afull, bias):
    bs = cin.shape[0]
    grid = bs // _NB

    def body(cin_ref, ct_ref, a_ref, b_ref, out_ref):
        ctv = ct_ref[...]
        parts = []
        for p in range(16):
            col = cin_ref[:, p : p + 1]
            oh = (col == lax.broadcasted_iota(jnp.int32, (_NB, 256), 1)).astype(
                jnp.float32
            )
            parts.append(jnp.dot(oh, ctv, preferred_element_type=jnp.float32))
        e = jnp.concatenate(parts, axis=1)
        y = jnp.dot(e, a_ref[...], preferred_element_type=jnp.float32)
        acc = y[:, 0:128]
        for t8 in range(1, 8):
            acc = jnp.maximum(acc, y[:, 128 * t8 : 128 * (t8 + 1)])
        pooled = jnp.maximum(acc[:, 0:64], acc[:, 64:128]) + b_ref[...]
        out_ref[...] = pooled

    return pl.pallas_call(
        body,
        grid=(grid,),
        in_specs=[
            pl.BlockSpec((_NB, 16), lambda i: (i, 0)),
            pl.BlockSpec((256, 64), lambda i: (0, 0)),
            pl.BlockSpec((1024, 1024), lambda i: (0, 0)),
            pl.BlockSpec((1, 64), lambda i: (0, 0)),
        ],
        out_specs=pl.BlockSpec((_NB, 64), lambda i: (i, 0)),
        out_shape=jax.ShapeDtypeStruct((bs, 64), jnp.float32),
    )(cin, ct, afull, bias)


_AB = 128


def _assemble_tc(wemb, cemb, b, s):
    ns = _AB * s
    grid = b // _AB

    def body(w_ref, c_ref, out_ref):
        out_ref[:, :, 0:128] = w_ref[...].reshape(_AB, s, 128)
        out_ref[:, :, 128:192] = c_ref[...].reshape(_AB, s, 64)

    return pl.pallas_call(
        body,
        grid=(grid,),
        in_specs=[
            pl.BlockSpec((ns, 128), lambda i: (i, 0)),
            pl.BlockSpec((ns, 64), lambda i: (i, 0)),
        ],
        out_specs=pl.BlockSpec((_AB, s, 192), lambda i: (i, 0, 0)),
        out_shape=jax.ShapeDtypeStruct((b, s, 192), jnp.float32),
    )(wemb, cemb)


def kernel(word_inputs, char_inputs, word_table, char_table, conv_w, conv_b):
    b, s = word_inputs.shape
    w = char_inputs.shape[-1]
    idx = word_inputs.reshape(-1).astype(jnp.int32)
    wemb = _sc_word_gather(word_table, idx)
    cin = char_inputs.reshape(-1, w).astype(jnp.int32)
    afull = _conv_matrix(conv_w)
    bias = conv_b.reshape(1, 64)
    cemb = _char_conv_tc(cin, char_table, afull, bias)
    return _assemble_tc(wemb, cemb, b, s)


# speedup vs baseline: 1.1446x; 1.0083x over previous
import functools

import numpy as np
import jax
import jax.numpy as jnp
from jax import lax
from jax.experimental import pallas as pl
from jax.experimental.pallas import tpu as pltpu
from jax.experimental.pallas import tpu_sc as plsc

_NC, _NS = 2, 16
_NW = _NC * _NS
_CH = 80


def _sc_word_gather(table, idx):
    (bs,) = idx.shape
    v, d = table.shape
    b_per_w = bs // _NW
    n_ch = b_per_w // _CH
    idx3 = idx.reshape(_NW, n_ch, _CH)
    mesh = plsc.VectorSubcoreMesh(core_axis_name="c", subcore_axis_name="s")

    @functools.partial(
        pl.kernel,
        mesh=mesh,
        compiler_params=pltpu.CompilerParams(use_tc_tiling_on_sc=True),
        out_type=jax.ShapeDtypeStruct((bs, d), jnp.float32),
        scratch_types=[
            pltpu.VMEM((n_ch, _CH), jnp.int32),
            pltpu.VMEM((2, _CH, d), jnp.float32),
            pltpu.SemaphoreType.DMA,
            pltpu.SemaphoreType.DMA,
        ],
    )
    def gather_kernel(table_hbm, idx_hbm, out_hbm, idx_v, rows_v, sem0, sem1):
        wid = lax.axis_index("s") * _NC + lax.axis_index("c")
        base = wid * b_per_w
        pltpu.sync_copy(idx_hbm.at[wid], idx_v)
        sems = (sem0, sem1)
        handles = {}

        def start(c):
            buf = c % 2
            handles[c] = pltpu.async_copy(
                table_hbm.at[idx_v.at[c]], rows_v.at[buf], sems[buf]
            )

        start(0)
        for c in range(n_ch):
            handles[c].wait()
            if c + 1 < n_ch:
                start(c + 1)
            pltpu.sync_copy(
                rows_v.at[c % 2], out_hbm.at[pl.ds(base + c * _CH, _CH)]
            )

    return gather_kernel(table, idx3)


def _conv_matrix(conv_w):
    wt = jnp.transpose(conv_w, (1, 0, 2))
    rowmod = (np.arange(1024) % 16).astype(np.int32)[:, None]
    colgrp = (np.arange(1024) // 64).astype(np.int32)[None, :]
    diff = jnp.asarray(rowmod - colgrp)
    a = jnp.zeros((1024, 1024), jnp.float32)
    for k in range(3):
        wexp = jnp.tile(jnp.repeat(wt[:, :, k], 16, axis=0), (1, 16))
        a = a + jnp.where(diff == (k - 1), wexp, 0.0)
    return a


_NB = 2048


def _char_conv_tc(cin, ct, afull, bias):
    bs = cin.shape[0]
    grid = bs // _NB

    def body(cin_ref, ct_ref, a_ref, b_ref, out_ref):
        ctv = ct_ref[...]
        parts = []
        for p in range(16):
            col = cin_ref[:, p : p + 1]
            oh = (col == lax.broadcasted_iota(jnp.int32, (_NB, 256), 1)).astype(
                jnp.float32
            )
            parts.append(jnp.dot(oh, ctv, preferred_element_type=jnp.float32))
        e = jnp.concatenate(parts, axis=1)
        y = jnp.dot(e, a_ref[...], preferred_element_type=jnp.float32)
        acc = y[:, 0:128]
        for t8 in range(1, 8):
            acc = jnp.maximum(acc, y[:, 128 * t8 : 128 * (t8 + 1)])
        pooled = jnp.maximum(acc[:, 0:64], acc[:, 64:128]) + b_ref[...]
        out_ref[...] = pooled

    return pl.pallas_call(
        body,
        grid=(grid,),
        in_specs=[
            pl.BlockSpec((_NB, 16), lambda i: (i, 0)),
            pl.BlockSpec((256, 64), lambda i: (0, 0)),
            pl.BlockSpec((1024, 1024), lambda i: (0, 0)),
            pl.BlockSpec((1, 64), lambda i: (0, 0)),
        ],
        out_specs=pl.BlockSpec((_NB, 64), lambda i: (i, 0)),
        out_shape=jax.ShapeDtypeStruct((bs, 64), jnp.float32),
    )(cin, ct, afull, bias)


_AB = 128


def _assemble_tc(wemb, cemb, b, s):
    ns = _AB * s
    grid = b // _AB

    def body(w_ref, c_ref, out_ref):
        out_ref[:, :, 0:128] = w_ref[...].reshape(_AB, s, 128)
        out_ref[:, :, 128:192] = c_ref[...].reshape(_AB, s, 64)

    return pl.pallas_call(
        body,
        grid=(grid,),
        in_specs=[
            pl.BlockSpec((ns, 128), lambda i: (i, 0)),
            pl.BlockSpec((ns, 64), lambda i: (i, 0)),
        ],
        out_specs=pl.BlockSpec((_AB, s, 192), lambda i: (i, 0, 0)),
        out_shape=jax.ShapeDtypeStruct((b, s, 192), jnp.float32),
    )(wemb, cemb)


def kernel(word_inputs, char_inputs, word_table, char_table, conv_w, conv_b):
    b, s = word_inputs.shape
    w = char_inputs.shape[-1]
    idx = word_inputs.reshape(-1).astype(jnp.int32)
    wemb = _sc_word_gather(word_table, idx)
    cin = char_inputs.reshape(-1, w).astype(jnp.int32)
    afull = _conv_matrix(conv_w)
    bias = conv_b.reshape(1, 64)
    cemb = _char_conv_tc(cin, char_table, afull, bias)
    return _assemble_tc(wemb, cemb, b, s)


# speedup vs baseline: 1.1534x; 1.0076x over previous
import functools

import numpy as np
import jax
import jax.numpy as jnp
from jax import lax
from jax.experimental import pallas as pl
from jax.experimental.pallas import tpu as pltpu
from jax.experimental.pallas import tpu_sc as plsc

_NC, _NS = 2, 16
_NW = _NC * _NS
_CH = 80


def _sc_word_gather(table, idx):
    (bs,) = idx.shape
    v, d = table.shape
    b_per_w = bs // _NW
    n_ch = b_per_w // _CH
    idx3 = idx.reshape(_NW, n_ch, _CH)
    mesh = plsc.VectorSubcoreMesh(core_axis_name="c", subcore_axis_name="s")

    @functools.partial(
        pl.kernel,
        mesh=mesh,
        compiler_params=pltpu.CompilerParams(use_tc_tiling_on_sc=True),
        out_type=jax.ShapeDtypeStruct((bs, d), jnp.float32),
        scratch_types=[
            pltpu.VMEM((n_ch, _CH), jnp.int32),
            pltpu.VMEM((2, _CH, d), jnp.float32),
            pltpu.SemaphoreType.DMA,
            pltpu.SemaphoreType.DMA,
        ],
    )
    def gather_kernel(table_hbm, idx_hbm, out_hbm, idx_v, rows_v, sem0, sem1):
        wid = lax.axis_index("s") * _NC + lax.axis_index("c")
        base = wid * b_per_w
        pltpu.sync_copy(idx_hbm.at[wid], idx_v)
        sems = (sem0, sem1)
        handles = {}

        def start(c):
            buf = c % 2
            handles[c] = pltpu.async_copy(
                table_hbm.at[idx_v.at[c]], rows_v.at[buf], sems[buf]
            )

        start(0)
        for c in range(n_ch):
            handles[c].wait()
            if c + 1 < n_ch:
                start(c + 1)
            pltpu.sync_copy(
                rows_v.at[c % 2], out_hbm.at[pl.ds(base + c * _CH, _CH)]
            )

    return gather_kernel(table, idx3)


def _conv_matrix(conv_w):
    wt = jnp.transpose(conv_w, (1, 0, 2))
    rowmod = (np.arange(1024) % 16).astype(np.int32)[:, None]
    colgrp = (np.arange(1024) // 64).astype(np.int32)[None, :]
    diff = jnp.asarray(rowmod - colgrp)
    a = jnp.zeros((1024, 1024), jnp.float32)
    for k in range(3):
        wexp = jnp.tile(jnp.repeat(wt[:, :, k], 16, axis=0), (1, 16))
        a = a + jnp.where(diff == (k - 1), wexp, 0.0)
    return a


_NB = 3200


def _char_conv_tc(cin, ct, afull, bias):
    bs = cin.shape[0]
    grid = bs // _NB

    def body(cin_ref, ct_ref, a_ref, b_ref, out_ref):
        ctv = ct_ref[...]
        parts = []
        for p in range(16):
            col = cin_ref[:, p : p + 1]
            oh = (col == lax.broadcasted_iota(jnp.int32, (_NB, 256), 1)).astype(
                jnp.float32
            )
            parts.append(jnp.dot(oh, ctv, preferred_element_type=jnp.float32))
        e = jnp.concatenate(parts, axis=1)
        y = jnp.dot(e, a_ref[...], preferred_element_type=jnp.float32)
        acc = y[:, 0:128]
        for t8 in range(1, 8):
            acc = jnp.maximum(acc, y[:, 128 * t8 : 128 * (t8 + 1)])
        pooled = jnp.maximum(acc[:, 0:64], acc[:, 64:128]) + b_ref[...]
        out_ref[...] = pooled

    return pl.pallas_call(
        body,
        grid=(grid,),
        in_specs=[
            pl.BlockSpec((_NB, 16), lambda i: (i, 0)),
            pl.BlockSpec((256, 64), lambda i: (0, 0)),
            pl.BlockSpec((1024, 1024), lambda i: (0, 0)),
            pl.BlockSpec((1, 64), lambda i: (0, 0)),
        ],
        out_specs=pl.BlockSpec((_NB, 64), lambda i: (i, 0)),
        out_shape=jax.ShapeDtypeStruct((bs, 64), jnp.float32),
    )(cin, ct, afull, bias)


_AB = 128


def _assemble_tc(wemb, cemb, b, s):
    ns = _AB * s
    grid = b // _AB

    def body(w_ref, c_ref, out_ref):
        out_ref[:, :, 0:128] = w_ref[...].reshape(_AB, s, 128)
        out_ref[:, :, 128:192] = c_ref[...].reshape(_AB, s, 64)

    return pl.pallas_call(
        body,
        grid=(grid,),
        in_specs=[
            pl.BlockSpec((ns, 128), lambda i: (i, 0)),
            pl.BlockSpec((ns, 64), lambda i: (i, 0)),
        ],
        out_specs=pl.BlockSpec((_AB, s, 192), lambda i: (i, 0, 0)),
        out_shape=jax.ShapeDtypeStruct((b, s, 192), jnp.float32),
    )(wemb, cemb)


def kernel(word_inputs, char_inputs, word_table, char_table, conv_w, conv_b):
    b, s = word_inputs.shape
    w = char_inputs.shape[-1]
    idx = word_inputs.reshape(-1).astype(jnp.int32)
    wemb = _sc_word_gather(word_table, idx)
    cin = char_inputs.reshape(-1, w).astype(jnp.int32)
    afull = _conv_matrix(conv_w)
    bias = conv_b.reshape(1, 64)
    cemb = _char_conv_tc(cin, char_table, afull, bias)
    return _assemble_tc(wemb, cemb, b, s)
